# baseline (speedup 1.0000x reference)
import functools

import jax
import jax.numpy as jnp
from jax import lax
from jax.experimental import pallas as pl
from jax.experimental.pallas import tpu as pltpu
from jax.experimental.pallas import tpu_sc as plsc

H = 64
SENT = 200
BATCH = 4096
VOCAB = 1000000
EPS = 1e-5
NC = 2
NS = 16
NW = NC * NS
SENT_PER_W = BATCH // NW
NCHUNK = SENT_PER_W
GROWS = SENT // 2
GTOT = BATCH * SENT // 2

_mesh = plsc.VectorSubcoreMesh(core_axis_name="c", subcore_axis_name="s")


@functools.partial(
    pl.kernel,
    out_type=jax.ShapeDtypeStruct((GTOT, 2 * H), jnp.float32),
    mesh=_mesh,
    scratch_types=[
        pltpu.VMEM((2, SENT), jnp.int32),
        pltpu.VMEM((2, SENT), jnp.int32),
        pltpu.VMEM((2, SENT, 2 * H), jnp.float32),
        pltpu.VMEM((2, GROWS, 2 * H), jnp.float32),
        pltpu.VMEM((SENT, H), jnp.float32),
        pltpu.VMEM((H,), jnp.float32),
        pltpu.VMEM((H,), jnp.float32),
        pltpu.SemaphoreType.DMA,
        pltpu.SemaphoreType.DMA,
        pltpu.SemaphoreType.DMA,
    ],
    compiler_params=pltpu.CompilerParams(
        needs_layout_passes=False, use_tc_tiling_on_sc=False),
)
def _sc_embed_ln(idsp_hbm, idsq_hbm, tok_hbm, pos_hbm, w_hbm, b_hbm, g_hbm,
                 idxp_v, idxq_v, rows_v, pair_v, pos_v, w_v, b_v,
                 gsem, osem, isem):
    cid = lax.axis_index("c")
    sid = lax.axis_index("s")
    wid = sid * NC + cid
    sent_w = wid * SENT_PER_W

    pltpu.sync_copy(pos_hbm.at[pl.ds(0, SENT)], pos_v)
    pltpu.sync_copy(w_hbm, w_v)
    pltpu.sync_copy(b_hbm, b_v)

    def load_ids(c, b):
        s0 = sent_w + c
        pltpu.async_copy(idsp_hbm.at[s0], idxp_v.at[b], isem)
        pltpu.async_copy(idsq_hbm.at[s0], idxq_v.at[b], isem)

    def drain_ids(b):
        pltpu.make_async_copy(idsp_hbm.at[0], idxp_v.at[b], isem).wait()
        pltpu.make_async_copy(idsq_hbm.at[0], idxq_v.at[b], isem).wait()

    def issue_gathers(b):
        for off, n in ((0, 128), (128, 72)):
            pltpu.async_copy(
                tok_hbm.at[idxp_v.at[b, pl.ds(off, n)]],
                rows_v.at[b, pl.ds(off, n)], gsem)

    def drain_gathers(b):
        pltpu.make_async_copy(
            tok_hbm.at[idxp_v.at[b]], rows_v.at[b], gsem).wait()

    def drain_out():
        pltpu.make_async_copy(
            pair_v.at[0], g_hbm.at[pl.ds(0, GROWS)], osem).wait()

    def ln_row(x, perms, wgt, bia):
        ss = (x[0] + x[1]) + (x[2] + x[3])
        q = (x[0] * x[0] + x[1] * x[1]) + (x[2] * x[2] + x[3] * x[3])
        for perm in perms:
            ss = ss + ss.at[perm].get(mode="promise_in_bounds")
            q = q + q.at[perm].get(mode="promise_in_bounds")
        mv = ss * (1.0 / H)
        vv = q * (1.0 / H) - mv * mv + EPS
        iv = plsc.bitcast(vv, jnp.int32)
        y = plsc.bitcast(jnp.int32(0x5F3759DF) - (iv >> 1), jnp.float32)
        hv = vv * 0.5
        y = y * (1.5 - hv * y * y)
        y = y * (1.5 - hv * y * y)
        my = mv * y
        return [(x[h] * y - my) * wgt[h] + bia[h] for h in range(4)]

    def compute(b):
        lanes = lax.iota(jnp.int32, 16)
        perms = [lanes ^ m for m in (1, 2, 4, 8)]
        wgt = [w_v[pl.ds(16 * h, 16)] for h in range(4)]
        bia = [b_v[pl.ds(16 * h, 16)] for h in range(4)]

        bsplat = jnp.full((16,), b, jnp.int32)

        @plsc.parallel_loop(0, GROWS, 1, unroll=2)
        def pair_loop(r2):
            for half in range(2):
                r = r2 + half * GROWS
                rsplat = jnp.broadcast_to(r, (16,)).astype(jnp.int32)
                qv = plsc.load_gather(idxq_v, [bsplat, rsplat])
                base = qv + lanes
                x = []
                for h in range(4):
                    xg = plsc.load_gather(rows_v, [bsplat, rsplat,
                                                   base + 16 * h])
                    x.append(xg + pos_v[r, pl.ds(16 * h, 16)])
                o = ln_row(x, perms, wgt, bia)
                for h in range(4):
                    pair_v[b, r2, pl.ds(half * H + 16 * h, 16)] = o[h]

    load_ids(0, 0)
    drain_ids(0)
    issue_gathers(0)
    load_ids(1, 1)

    @pl.loop(0, NCHUNK // 2)
    def main_loop(t):
        for b in range(2):
            c = t * 2 + b
            nb = 1 - b

            @pl.when(c + 1 < NCHUNK)
            def _():
                @pl.when(c >= 1)
                def _():
                    drain_out()
                drain_ids(nb)
                issue_gathers(nb)

            @pl.when(c + 2 < NCHUNK)
            def _():
                load_ids(c + 2, b)

            drain_gathers(b)
            compute(b)
            pltpu.async_copy(
                pair_v.at[b],
                g_hbm.at[pl.ds(wid * (SENT_PER_W * GROWS) + c * GROWS,
                               GROWS)], osem)

    drain_out()
    drain_out()


SB = 16


def _tc_finish_body(g_ref, out_ref):
    y = g_ref[...].reshape(SB, SENT // 2, 2 * H)
    out_ref[:, : SENT // 2, :] = y[:, :, :H]
    out_ref[:, SENT // 2:, :] = y[:, :, H:]


_tc_finish = pl.pallas_call(
    _tc_finish_body,
    out_shape=jax.ShapeDtypeStruct((BATCH, SENT, H), jnp.float32),
    grid=(BATCH // SB,),
    in_specs=[pl.BlockSpec((SB * SENT // 2, 2 * H), lambda i: (i, 0))],
    out_specs=pl.BlockSpec((SB, SENT, H), lambda i: (i, 0, 0)),
)


def kernel(batch_token_ids, token_table, pos_table, ln_weight, ln_bias):
    ids = batch_token_ids.astype(jnp.int32)
    tok_pair = token_table.reshape(VOCAB // 2, 2 * H)
    ids_p = ids >> 1
    ids_q = (ids & 1) * H
    g = _sc_embed_ln(ids_p, ids_q, tok_pair, pos_table,
                     ln_weight, ln_bias)
    return _tc_finish(g)


# speedup vs baseline: 1.1131x; 1.1131x over previous
import functools

import jax
import jax.numpy as jnp
from jax import lax
from jax.experimental import pallas as pl
from jax.experimental.pallas import tpu as pltpu
from jax.experimental.pallas import tpu_sc as plsc

H = 64
SENT = 200
BATCH = 4096
EPS = 1e-5
NC = 2
NS = 16
NW = NC * NS
SPC = 2
CHUNK = SPC * SENT
SENT_PER_W = BATCH // NW
NCHUNK = SENT_PER_W // SPC

_mesh = plsc.VectorSubcoreMesh(core_axis_name="c", subcore_axis_name="s")


@functools.partial(
    pl.kernel,
    out_type=jax.ShapeDtypeStruct((BATCH, SENT, H), jnp.float32),
    mesh=_mesh,
    scratch_types=[
        pltpu.VMEM((2, SPC, SENT), jnp.int32),
        pltpu.VMEM((2, SPC, SENT, H), jnp.float32),
        pltpu.VMEM((SENT, H), jnp.float32),
        pltpu.VMEM((H,), jnp.float32),
        pltpu.VMEM((H,), jnp.float32),
        pltpu.SemaphoreType.DMA,
        pltpu.SemaphoreType.DMA,
    ],
    compiler_params=pltpu.CompilerParams(
        needs_layout_passes=False, use_tc_tiling_on_sc=False),
)
def _sc_embed_ln(ids_hbm, tok_hbm, pos_hbm, w_hbm, b_hbm, out_hbm,
                 idx_v, rows_v, pos_v, w_v, b_v, gsem, osem):
    cid = lax.axis_index("c")
    sid = lax.axis_index("s")
    wid = sid * NC + cid
    sent_w = wid * SENT_PER_W

    pltpu.sync_copy(pos_hbm.at[pl.ds(0, SENT)], pos_v)
    pltpu.sync_copy(w_hbm, w_v)
    pltpu.sync_copy(b_hbm, b_v)

    def issue(c, b):
        s0 = sent_w + c * SPC
        pltpu.sync_copy(ids_hbm.at[pl.ds(s0, SPC)], idx_v.at[b])
        for s in range(SPC):
            for off, n in ((0, 128), (128, 72)):
                pltpu.async_copy(
                    tok_hbm.at[idx_v.at[b, s, pl.ds(off, n)]],
                    rows_v.at[b, s, pl.ds(off, n)], gsem)

    def drain_gathers(b):
        pltpu.make_async_copy(
            tok_hbm.at[idx_v.at[b, 0]], rows_v.at[b, 0], gsem).wait()
        pltpu.make_async_copy(
            tok_hbm.at[idx_v.at[b, 1]], rows_v.at[b, 1], gsem).wait()

    def drain_out():
        pltpu.make_async_copy(
            rows_v.at[0], out_hbm.at[pl.ds(0, SPC)], osem).wait()

    def compute(b):
        lanes = lax.iota(jnp.int32, 16)
        perms = [lanes ^ m for m in (1, 2, 4, 8)]
        wgt = [w_v[pl.ds(16 * h, 16)] for h in range(4)]
        bia = [b_v[pl.ds(16 * h, 16)] for h in range(4)]

        for s in range(SPC):
            @plsc.parallel_loop(0, SENT, 1, unroll=4)
            def row_loop(r):
                x = []
                for h in range(4):
                    x.append(rows_v[b, s, r, pl.ds(16 * h, 16)]
                             + pos_v[r, pl.ds(16 * h, 16)])
                ss = (x[0] + x[1]) + (x[2] + x[3])
                q = (x[0] * x[0] + x[1] * x[1]) + (x[2] * x[2] + x[3] * x[3])
                for perm in perms:
                    ss = ss + ss.at[perm].get(mode="promise_in_bounds")
                    q = q + q.at[perm].get(mode="promise_in_bounds")
                mv = ss * (1.0 / H)
                vv = q * (1.0 / H) - mv * mv + EPS
                iv = plsc.bitcast(vv, jnp.int32)
                y = plsc.bitcast(
                    jnp.int32(0x5F3759DF) - (iv >> 1), jnp.float32)
                hv = vv * 0.5
                y = y * (1.5 - hv * y * y)
                y = y * (1.5 - hv * y * y)
                my = mv * y
                for h in range(4):
                    rows_v[b, s, r, pl.ds(16 * h, 16)] = (
                        (x[h] * y - my) * wgt[h] + bia[h])

    issue(0, 0)

    @pl.loop(0, NCHUNK // 2)
    def pair_loop(t):
        for b in range(2):
            c = t * 2 + b
            nb = 1 - b

            @pl.when(c + 1 < NCHUNK)
            def _():
                @pl.when(c >= 1)
                def _():
                    drain_out()
                issue(c + 1, nb)

            drain_gathers(b)
            compute(b)
            pltpu.async_copy(
                rows_v.at[b],
                out_hbm.at[pl.ds(sent_w + c * SPC, SPC)], osem)

    drain_out()
    drain_out()


def kernel(batch_token_ids, token_table, pos_table, ln_weight, ln_bias):
    ids = batch_token_ids.astype(jnp.int32)
    return _sc_embed_ln(ids, token_table, pos_table, ln_weight, ln_bias)


# speedup vs baseline: 1.1195x; 1.0058x over previous
import functools

import jax
import jax.numpy as jnp
from jax import lax
from jax.experimental import pallas as pl
from jax.experimental.pallas import tpu as pltpu
from jax.experimental.pallas import tpu_sc as plsc

H = 64
SENT = 200
BATCH = 4096
EPS = 1e-5
NC = 2
NS = 16
NW = NC * NS
SPC = 4
CHUNK = SPC * SENT
SENT_PER_W = BATCH // NW
NCHUNK = SENT_PER_W // SPC

_mesh = plsc.VectorSubcoreMesh(core_axis_name="c", subcore_axis_name="s")


@functools.partial(
    pl.kernel,
    out_type=jax.ShapeDtypeStruct((BATCH, SENT, H), jnp.float32),
    mesh=_mesh,
    scratch_types=[
        pltpu.VMEM((2, SPC, SENT), jnp.int32),
        pltpu.VMEM((2, SPC, SENT, H), jnp.float32),
        pltpu.VMEM((SENT, H), jnp.float32),
        pltpu.VMEM((H,), jnp.float32),
        pltpu.VMEM((H,), jnp.float32),
        pltpu.SemaphoreType.DMA,
        pltpu.SemaphoreType.DMA,
    ],
    compiler_params=pltpu.CompilerParams(
        needs_layout_passes=False, use_tc_tiling_on_sc=False),
)
def _sc_embed_ln(ids_hbm, tok_hbm, pos_hbm, w_hbm, b_hbm, out_hbm,
                 idx_v, rows_v, pos_v, w_v, b_v, gsem, osem):
    cid = lax.axis_index("c")
    sid = lax.axis_index("s")
    wid = sid * NC + cid
    sent_w = wid * SENT_PER_W

    pltpu.sync_copy(pos_hbm.at[pl.ds(0, SENT)], pos_v)
    pltpu.sync_copy(w_hbm, w_v)
    pltpu.sync_copy(b_hbm, b_v)

    def issue(c, b):
        s0 = sent_w + c * SPC
        pltpu.sync_copy(ids_hbm.at[pl.ds(s0, SPC)], idx_v.at[b])
        for s in range(SPC):
            for off, n in ((0, 128), (128, 72)):
                pltpu.async_copy(
                    tok_hbm.at[idx_v.at[b, s, pl.ds(off, n)]],
                    rows_v.at[b, s, pl.ds(off, n)], gsem)

    def drain_gathers(b):
        for s in range(SPC):
            pltpu.make_async_copy(
                tok_hbm.at[idx_v.at[b, s]], rows_v.at[b, s], gsem).wait()

    def drain_out():
        pltpu.make_async_copy(
            rows_v.at[0], out_hbm.at[pl.ds(0, SPC)], osem).wait()

    def compute(b):
        lanes = lax.iota(jnp.int32, 16)
        perms = [lanes ^ m for m in (1, 2, 4, 8)]
        wgt = [w_v[pl.ds(16 * h, 16)] for h in range(4)]
        bia = [b_v[pl.ds(16 * h, 16)] for h in range(4)]

        for s in range(SPC):
            @plsc.parallel_loop(0, SENT, 1, unroll=4)
            def row_loop(r):
                x = []
                for h in range(4):
                    x.append(rows_v[b, s, r, pl.ds(16 * h, 16)]
                             + pos_v[r, pl.ds(16 * h, 16)])
                ss = (x[0] + x[1]) + (x[2] + x[3])
                q = (x[0] * x[0] + x[1] * x[1]) + (x[2] * x[2] + x[3] * x[3])
                for perm in perms:
                    ss = ss + ss.at[perm].get(mode="promise_in_bounds")
                    q = q + q.at[perm].get(mode="promise_in_bounds")
                mv = ss * (1.0 / H)
                vv = q * (1.0 / H) - mv * mv + EPS
                iv = plsc.bitcast(vv, jnp.int32)
                y = plsc.bitcast(
                    jnp.int32(0x5F3759DF) - (iv >> 1), jnp.float32)
                hv = vv * 0.5
                y = y * (1.5 - hv * y * y)
                y = y * (1.5 - hv * y * y)
                my = mv * y
                for h in range(4):
                    rows_v[b, s, r, pl.ds(16 * h, 16)] = (
                        (x[h] * y - my) * wgt[h] + bia[h])

    issue(0, 0)

    @pl.loop(0, NCHUNK // 2)
    def pair_loop(t):
        for b in range(2):
            c = t * 2 + b
            nb = 1 - b

            @pl.when(c + 1 < NCHUNK)
            def _():
                @pl.when(c >= 1)
                def _():
                    drain_out()
                issue(c + 1, nb)

            drain_gathers(b)
            compute(b)
            pltpu.async_copy(
                rows_v.at[b],
                out_hbm.at[pl.ds(sent_w + c * SPC, SPC)], osem)

    drain_out()
    drain_out()


def kernel(batch_token_ids, token_table, pos_table, ln_weight, ln_bias):
    ids = batch_token_ids.astype(jnp.int32)
    return _sc_embed_ln(ids, token_table, pos_table, ln_weight, ln_bias)
